# baseline (speedup 1.0000x reference)
import jax
import jax.numpy as jnp
from jax.experimental import pallas as pl
from jax.experimental.pallas import tpu as pltpu


def _soft_argmax_block(a, xs, ys):
    colsum = a.sum(axis=1)
    rowsum = a.sum(axis=2)
    zeta = colsum.sum(axis=1)
    kx = jnp.round((colsum * xs).sum(axis=1) / zeta)
    ky = jnp.round((rowsum * ys).sum(axis=1) / zeta)
    return zeta, kx, ky


def _kernel(hm_ref, tf_ref, map_ref, zeta_ref, kp_ref, tf_kp_ref):
    a = jnp.abs(hm_ref[0])
    map_ref[0] = a
    C, H, W = a.shape
    xs = jax.lax.broadcasted_iota(jnp.int32, (C, W), 1).astype(jnp.float32)
    ys = jax.lax.broadcasted_iota(jnp.int32, (C, H), 1).astype(jnp.float32)

    zeta, kx, ky = _soft_argmax_block(a, xs, ys)
    zeta_ref[0, 0, :] = zeta
    kp_ref[0, 0, :] = kx
    kp_ref[0, 1, :] = ky

    t = jnp.abs(tf_ref[0])
    _, tkx, tky = _soft_argmax_block(t, xs, ys)
    tf_kp_ref[0, 0, :] = tkx
    tf_kp_ref[0, 1, :] = tky


def kernel(combined_hm_preds, tf_combined_hm_preds, cur_batch):
    B, C, H, W = combined_hm_preds.shape
    in_spec = pl.BlockSpec((1, C, H, W), lambda b: (b, 0, 0, 0))
    map_val, zeta, kp, tf_kp = pl.pallas_call(
        _kernel,
        grid=(B,),
        in_specs=[in_spec, in_spec],
        out_specs=(
            pl.BlockSpec((1, C, H, W), lambda b: (b, 0, 0, 0)),
            pl.BlockSpec((1, 1, C), lambda b: (b, 0, 0)),
            pl.BlockSpec((1, 2, C), lambda b: (b, 0, 0)),
            pl.BlockSpec((1, 2, C), lambda b: (b, 0, 0)),
        ),
        out_shape=(
            jax.ShapeDtypeStruct((B, C, H, W), jnp.float32),
            jax.ShapeDtypeStruct((B, 1, C), jnp.float32),
            jax.ShapeDtypeStruct((B, 2, C), jnp.float32),
            jax.ShapeDtypeStruct((B, 2, C), jnp.float32),
        ),
        compiler_params=pltpu.CompilerParams(
            dimension_semantics=("parallel",),
            vmem_limit_bytes=56 * 1024 * 1024,
        ),
    )(combined_hm_preds, tf_combined_hm_preds)
    keypoint = kp.transpose(0, 2, 1)
    tf_keypoint = tf_kp.transpose(0, 2, 1)
    return (map_val, keypoint, zeta.reshape(B, C), tf_keypoint)
